# baseline (speedup 1.0000x reference)
import functools

import jax
import jax.numpy as jnp
from jax import lax
from jax.experimental import pallas as pl
from jax.experimental.pallas import tpu as pltpu
from jax.experimental.pallas import tpu_sc as plsc

_F32_MAX = 3.0e38


def _dist_body(NI, NJ, TR, TC, CH,
               ref_blk, srcT_blk,
               fwd_min_o, fwd_idx_o, bwd_min_o, bwd_idx_o,
               A, I, bmin_s, bidx_s):
    i = pl.program_id(0)
    j = pl.program_id(1)
    NCH = TR // CH

    bx = srcT_blk[0:1, :]
    by = srcT_blk[1:2, :]
    bz = srcT_blk[2:3, :]

    colf = (jax.lax.broadcasted_iota(jnp.int32, (1, TC), 1).astype(jnp.float32)
            + jnp.float32(TC) * j.astype(jnp.float32))

    @pl.when(j == 0)
    def _():
        A[...] = jnp.full((TR, TC), _F32_MAX, jnp.float32)

    @pl.when(i == 0)
    def _():
        bmin_s[j] = jnp.full((2, CH, TC), _F32_MAX, jnp.float32)

    rowc = jax.lax.broadcasted_iota(jnp.int32, (CH, 1), 0).astype(jnp.float32)
    colf_b = jnp.broadcast_to(colf, (CH, TC))

    for r in range(NCH):
        a = ref_blk[r * CH:(r + 1) * CH, :]
        ax = a[:, 0:1]
        ay = a[:, 1:2]
        az = a[:, 2:3]
        dx = ax - bx
        dy = ay - by
        dz = az - bz
        d2 = dx * dx + dy * dy + dz * dz

        sl = slice(r * CH, (r + 1) * CH)
        Ac = A[sl, :]
        m = d2 < Ac
        A[sl, :] = jnp.where(m, d2, Ac)
        pltpu.store(I.at[sl, :], colf_b, mask=m)

        p = r % 2
        Bv = bmin_s[j, p]
        m2 = d2 < Bv
        bmin_s[j, p] = jnp.where(m2, d2, Bv)
        pltpu.store(bidx_s.at[j, p],
                    jnp.full((CH, TC), float(r), jnp.float32), mask=m2)

    @pl.when(j == NJ - 1)
    def _():
        Af = A[...]
        rm = jnp.min(Af, axis=1, keepdims=True)
        cand = jnp.where(Af == rm, I[...], _F32_MAX)
        fwd_min_o[...] = jnp.sqrt(rm)
        fwd_idx_o[...] = jnp.min(cand, axis=1, keepdims=True).astype(jnp.int32)

    @pl.when(i == NI - 1)
    def _():
        r0 = bidx_s[j, 0] * jnp.float32(CH) + rowc
        r1 = bidx_s[j, 1] * jnp.float32(CH) + rowc
        B0 = bmin_s[j, 0]
        B1 = bmin_s[j, 1]
        take1 = (B1 < B0) | ((B1 == B0) & (r1 < r0))
        Bf = jnp.where(take1, B1, B0)
        rows = jnp.where(take1, r1, r0)
        cm = jnp.min(Bf, axis=0, keepdims=True)
        candr = jnp.where(Bf == cm, rows, _F32_MAX)
        bwd_min_o[...] = jnp.sqrt(cm)
        bwd_idx_o[...] = jnp.min(candr, axis=0, keepdims=True).astype(jnp.int32)


def _min_argmin_both(ref_kpts, srcT):
    N = ref_kpts.shape[0]
    M = srcT.shape[1]
    TR = min(8192, N)
    TC = min(512, M)
    CH = min(16, TR)
    NI = N // TR
    NJ = M // TC

    body = functools.partial(_dist_body, NI, NJ, TR, TC, CH)
    return pl.pallas_call(
        body,
        grid=(NI, NJ),
        in_specs=[
            pl.BlockSpec((TR, 3), lambda i, j: (i, 0)),
            pl.BlockSpec((3, TC), lambda i, j: (0, j)),
        ],
        out_specs=[
            pl.BlockSpec((TR, 1), lambda i, j: (i, 0)),
            pl.BlockSpec((TR, 1), lambda i, j: (i, 0)),
            pl.BlockSpec((1, TC), lambda i, j: (0, j)),
            pl.BlockSpec((1, TC), lambda i, j: (0, j)),
        ],
        out_shape=[
            jax.ShapeDtypeStruct((N, 1), jnp.float32),
            jax.ShapeDtypeStruct((N, 1), jnp.int32),
            jax.ShapeDtypeStruct((1, M), jnp.float32),
            jax.ShapeDtypeStruct((1, M), jnp.int32),
        ],
        scratch_shapes=[
            pltpu.VMEM((TR, TC), jnp.float32),
            pltpu.VMEM((TR, TC), jnp.float32),
            pltpu.VMEM((NJ, 2, CH, TC), jnp.float32),
            pltpu.VMEM((NJ, 2, CH, TC), jnp.float32),
        ],
        compiler_params=pltpu.CompilerParams(
            dimension_semantics=("arbitrary", "arbitrary"),
        ),
    )(ref_kpts, srcT)


def _sc_gather_sigmas(src_sigma, ref_sigma, fwd_idx, bwd_idx):
    N = fwd_idx.shape[0]
    M = bwd_idx.shape[0]
    info = plsc.get_sparse_core_info()
    NW = info.num_cores * info.num_subcores
    bf = N // NW
    bb = M // NW
    mesh = plsc.VectorSubcoreMesh(core_axis_name="c", subcore_axis_name="s")

    @functools.partial(
        pl.kernel, mesh=mesh,
        out_type=[jax.ShapeDtypeStruct((N,), jnp.float32),
                  jax.ShapeDtypeStruct((M,), jnp.float32)],
        scratch_types=[
            pltpu.VMEM((bf,), jnp.int32),
            pltpu.VMEM((bf,), jnp.float32),
            pltpu.VMEM((bb,), jnp.int32),
            pltpu.VMEM((bb,), jnp.float32),
            pltpu.SemaphoreType.DMA,
        ],
    )
    def k(src_sig_hbm, ref_sig_hbm, fidx_hbm, bidx_hbm,
          sel2_hbm, sel1_hbm, fidx_v, fsel_v, bidx_v, bsel_v, sem):
        wid = lax.axis_index("s") * info.num_cores + lax.axis_index("c")
        base_f = wid * bf
        pltpu.sync_copy(fidx_hbm.at[pl.ds(base_f, bf)], fidx_v)
        pltpu.async_copy(src_sig_hbm.at[fidx_v], fsel_v, sem).wait()
        pltpu.sync_copy(fsel_v, sel2_hbm.at[pl.ds(base_f, bf)])

        base_b = wid * bb
        pltpu.sync_copy(bidx_hbm.at[pl.ds(base_b, bb)], bidx_v)
        pltpu.async_copy(ref_sig_hbm.at[bidx_v], bsel_v, sem).wait()
        pltpu.sync_copy(bsel_v, sel1_hbm.at[pl.ds(base_b, bb)])

    return k(src_sigma, ref_sigma, fwd_idx, bwd_idx)


def _loss_body(n_f, n_b, fm, sig_a_f, sig_sel_f, bm, sig_a_b, sig_sel_b, out):
    sf = (sig_a_f[...] + sig_sel_f[...]) * 0.5
    sb = (sig_a_b[...] + sig_sel_b[...]) * 0.5
    tf = jnp.log(sf) + fm[...] / sf
    tb = jnp.log(sb) + bm[...] / sb
    out[0, 0] = jnp.sum(tf) * (1.0 / n_f) + jnp.sum(tb) * (1.0 / n_b)


def _loss_finish(fm, ref_sigma, sel2, bm, src_sigma, sel1):
    n_f = fm.size
    n_b = bm.size
    shp_f = (n_f // 128, 128)
    shp_b = (n_b // 128, 128)
    args = [fm.reshape(shp_f), ref_sigma.reshape(shp_f), sel2.reshape(shp_f),
            bm.reshape(shp_b), src_sigma.reshape(shp_b), sel1.reshape(shp_b)]
    out = pl.pallas_call(
        functools.partial(_loss_body, float(n_f), float(n_b)),
        out_specs=pl.BlockSpec(memory_space=pltpu.SMEM),
        out_shape=jax.ShapeDtypeStruct((1, 1), jnp.float32),
    )(*args)
    return out[0, 0]


def kernel(ref_kpts, src_kpts, gt_transform, ref_sigma, src_sigma):
    keypoints2 = src_kpts @ gt_transform[:3, :3].T + gt_transform[:3, 3]
    srcT = keypoints2.T
    fm, fi, bm, bi = _min_argmin_both(ref_kpts, srcT)
    sel2, sel1 = _sc_gather_sigmas(src_sigma, ref_sigma,
                                   fi.reshape(-1), bi.reshape(-1))
    return _loss_finish(fm.reshape(-1), ref_sigma, sel2,
                        bm.reshape(-1), src_sigma, sel1)


# speedup vs baseline: 1.0321x; 1.0321x over previous
import functools

import jax
import jax.numpy as jnp
from jax import lax
from jax.experimental import pallas as pl
from jax.experimental.pallas import tpu as pltpu
from jax.experimental.pallas import tpu_sc as plsc

_F32_MAX = 3.0e38


def _dist_body(NI, NJ, TR, TC, CH,
               ref_blk, srcT_blk,
               fwd_min_o, fwd_idx_o, bwd_min_o, bwd_idx_o,
               A, I, bmin_s, bidx_s):
    i = pl.program_id(0)
    j = pl.program_id(1)
    NCH = TR // CH

    bx = srcT_blk[0:1, :]
    by = srcT_blk[1:2, :]
    bz = srcT_blk[2:3, :]

    colf = (jax.lax.broadcasted_iota(jnp.int32, (1, TC), 1).astype(jnp.float32)
            + jnp.float32(TC) * j.astype(jnp.float32))

    @pl.when(j == 0)
    def _():
        A[...] = jnp.full((TR, TC), _F32_MAX, jnp.float32)

    @pl.when(i == 0)
    def _():
        bmin_s[j] = jnp.full((CH, TC), _F32_MAX, jnp.float32)

    rowc = jax.lax.broadcasted_iota(jnp.int32, (CH, 1), 0).astype(jnp.float32)
    colf_b = jnp.broadcast_to(colf, (CH, TC))

    for r in range(NCH):
        a = ref_blk[r * CH:(r + 1) * CH, :]
        ax = a[:, 0:1]
        ay = a[:, 1:2]
        az = a[:, 2:3]
        dx = ax - bx
        dy = ay - by
        dz = az - bz
        d2 = dx * dx + dy * dy + dz * dz

        sl = slice(r * CH, (r + 1) * CH)
        Ac = A[sl, :]
        m = d2 < Ac
        A[sl, :] = jnp.where(m, d2, Ac)
        pltpu.store(I.at[sl, :], colf_b, mask=m)

        Bv = bmin_s[j]
        m2 = d2 < Bv
        bmin_s[j] = jnp.where(m2, d2, Bv)
        pltpu.store(bidx_s.at[j],
                    jnp.full((CH, TC), float(r), jnp.float32), mask=m2)

    @pl.when(j == NJ - 1)
    def _():
        Af = A[...]
        rm = jnp.min(Af, axis=1, keepdims=True)
        cand = jnp.where(Af == rm, I[...], _F32_MAX)
        fwd_min_o[...] = jnp.sqrt(rm)
        fwd_idx_o[...] = jnp.min(cand, axis=1, keepdims=True).astype(jnp.int32)

    @pl.when(i == NI - 1)
    def _():
        Bf = bmin_s[j]
        cm = jnp.min(Bf, axis=0, keepdims=True)
        rows = bidx_s[j] * jnp.float32(CH) + rowc
        candr = jnp.where(Bf == cm, rows, _F32_MAX)
        bwd_min_o[...] = jnp.sqrt(cm)
        bwd_idx_o[...] = jnp.min(candr, axis=0, keepdims=True).astype(jnp.int32)


def _min_argmin_both(ref_kpts, srcT):
    N = ref_kpts.shape[0]
    M = srcT.shape[1]
    TR = min(8192, N)
    TC = min(512, M)
    CH = min(16, TR)
    NI = N // TR
    NJ = M // TC

    body = functools.partial(_dist_body, NI, NJ, TR, TC, CH)
    return pl.pallas_call(
        body,
        grid=(NI, NJ),
        in_specs=[
            pl.BlockSpec((TR, 3), lambda i, j: (i, 0)),
            pl.BlockSpec((3, TC), lambda i, j: (0, j)),
        ],
        out_specs=[
            pl.BlockSpec((TR, 1), lambda i, j: (i, 0)),
            pl.BlockSpec((TR, 1), lambda i, j: (i, 0)),
            pl.BlockSpec((1, TC), lambda i, j: (0, j)),
            pl.BlockSpec((1, TC), lambda i, j: (0, j)),
        ],
        out_shape=[
            jax.ShapeDtypeStruct((N, 1), jnp.float32),
            jax.ShapeDtypeStruct((N, 1), jnp.int32),
            jax.ShapeDtypeStruct((1, M), jnp.float32),
            jax.ShapeDtypeStruct((1, M), jnp.int32),
        ],
        scratch_shapes=[
            pltpu.VMEM((TR, TC), jnp.float32),
            pltpu.VMEM((TR, TC), jnp.float32),
            pltpu.VMEM((NJ, CH, TC), jnp.float32),
            pltpu.VMEM((NJ, CH, TC), jnp.float32),
        ],
        compiler_params=pltpu.CompilerParams(
            dimension_semantics=("arbitrary", "arbitrary"),
        ),
    )(ref_kpts, srcT)


def _sc_gather_sigmas(src_sigma, ref_sigma, fwd_idx, bwd_idx):
    N = fwd_idx.shape[0]
    M = bwd_idx.shape[0]
    info = plsc.get_sparse_core_info()
    NW = info.num_cores * info.num_subcores
    bf = N // NW
    bb = M // NW
    mesh = plsc.VectorSubcoreMesh(core_axis_name="c", subcore_axis_name="s")

    @functools.partial(
        pl.kernel, mesh=mesh,
        out_type=[jax.ShapeDtypeStruct((N,), jnp.float32),
                  jax.ShapeDtypeStruct((M,), jnp.float32)],
        scratch_types=[
            pltpu.VMEM((bf,), jnp.int32),
            pltpu.VMEM((bf,), jnp.float32),
            pltpu.VMEM((bb,), jnp.int32),
            pltpu.VMEM((bb,), jnp.float32),
            pltpu.SemaphoreType.DMA,
        ],
    )
    def k(src_sig_hbm, ref_sig_hbm, fidx_hbm, bidx_hbm,
          sel2_hbm, sel1_hbm, fidx_v, fsel_v, bidx_v, bsel_v, sem):
        wid = lax.axis_index("s") * info.num_cores + lax.axis_index("c")
        base_f = wid * bf
        pltpu.sync_copy(fidx_hbm.at[pl.ds(base_f, bf)], fidx_v)
        pltpu.async_copy(src_sig_hbm.at[fidx_v], fsel_v, sem).wait()
        pltpu.sync_copy(fsel_v, sel2_hbm.at[pl.ds(base_f, bf)])

        base_b = wid * bb
        pltpu.sync_copy(bidx_hbm.at[pl.ds(base_b, bb)], bidx_v)
        pltpu.async_copy(ref_sig_hbm.at[bidx_v], bsel_v, sem).wait()
        pltpu.sync_copy(bsel_v, sel1_hbm.at[pl.ds(base_b, bb)])

    return k(src_sigma, ref_sigma, fwd_idx, bwd_idx)


def _loss_body(n_f, n_b, fm, sig_a_f, sig_sel_f, bm, sig_a_b, sig_sel_b, out):
    sf = (sig_a_f[...] + sig_sel_f[...]) * 0.5
    sb = (sig_a_b[...] + sig_sel_b[...]) * 0.5
    tf = jnp.log(sf) + fm[...] / sf
    tb = jnp.log(sb) + bm[...] / sb
    out[0, 0] = jnp.sum(tf) * (1.0 / n_f) + jnp.sum(tb) * (1.0 / n_b)


def _loss_finish(fm, ref_sigma, sel2, bm, src_sigma, sel1):
    n_f = fm.size
    n_b = bm.size
    shp_f = (n_f // 128, 128)
    shp_b = (n_b // 128, 128)
    args = [fm.reshape(shp_f), ref_sigma.reshape(shp_f), sel2.reshape(shp_f),
            bm.reshape(shp_b), src_sigma.reshape(shp_b), sel1.reshape(shp_b)]
    out = pl.pallas_call(
        functools.partial(_loss_body, float(n_f), float(n_b)),
        out_specs=pl.BlockSpec(memory_space=pltpu.SMEM),
        out_shape=jax.ShapeDtypeStruct((1, 1), jnp.float32),
    )(*args)
    return out[0, 0]


def kernel(ref_kpts, src_kpts, gt_transform, ref_sigma, src_sigma):
    keypoints2 = src_kpts @ gt_transform[:3, :3].T + gt_transform[:3, 3]
    srcT = keypoints2.T
    fm, fi, bm, bi = _min_argmin_both(ref_kpts, srcT)
    sel2, sel1 = _sc_gather_sigmas(src_sigma, ref_sigma,
                                   fi.reshape(-1), bi.reshape(-1))
    return _loss_finish(fm.reshape(-1), ref_sigma, sel2,
                        bm.reshape(-1), src_sigma, sel1)
